# Initial kernel scaffold; baseline (speedup 1.0000x reference)
import functools

import jax
import jax.numpy as jnp
from jax import lax
from jax.experimental import pallas as pl
from jax.experimental.pallas import tpu as pltpu
from jax.experimental.pallas import tpu_sc as plsc

N_NODES = 10000
D_FEAT = 128
HIDDEN = 128
N_EDGES = 320000

_NC = 2
_NS = 16
_NW = _NC * _NS
_EPW = N_EDGES // _NW
_CHUNK = 80
_NCHUNK = _EPW // _CHUNK
_STRIPE = N_NODES // _NS
_ZROWS = 125
_CNTW = 16

_ROWBLK = 400
_NBLK = N_NODES // _ROWBLK


def _seg_body(with_cnt, y_hbm, src_hbm, dst_hbm, seg_out, *rest):
    if with_cnt:
        (cnt_out, acc_sh, cnt_sh, src_v, dst_v, rows_v, ones_v, zbuf_v,
         zc_v, sem) = rest
    else:
        acc_sh, src_v, dst_v, rows_v, zbuf_v, sem = rest
    cid = lax.axis_index("c")
    sid = lax.axis_index("s")
    wid = cid * _NS + sid
    base = sid * _STRIPE

    def _zb(i, c):
        zbuf_v[i // 8, pl.ds((i % 8) * 16, 16)] = jnp.zeros((16,), jnp.float32)
        return c
    lax.fori_loop(0, _ZROWS * 8, _zb, 0)
    for k in range(_STRIPE // _ZROWS):
        pltpu.sync_copy(zbuf_v, acc_sh.at[pl.ds(base + k * _ZROWS, _ZROWS)])

    if with_cnt:
        def _zc(i, c):
            zc_v[i, :] = jnp.zeros((16,), jnp.float32)
            return c
        lax.fori_loop(0, _ZROWS, _zc, 0)
        for k in range(_STRIPE // _ZROWS):
            pltpu.sync_copy(zc_v, cnt_sh.at[pl.ds(base + k * _ZROWS, _ZROWS)])

        def _oc(i, c):
            ones_v[i, :] = jnp.ones((16,), jnp.float32)
            return c
        lax.fori_loop(0, _CHUNK, _oc, 0)

    pltpu.sync_copy(src_hbm.at[wid], src_v)
    pltpu.sync_copy(dst_hbm.at[wid], dst_v)

    plsc.subcore_barrier()

    def _step(j, c):
        pltpu.async_copy(y_hbm.at[src_v.at[j]], rows_v, sem).wait()
        pltpu.sync_copy(rows_v, acc_sh.at[dst_v.at[j]], add=True)
        if with_cnt:
            pltpu.sync_copy(ones_v, cnt_sh.at[dst_v.at[j]], add=True)
        return c
    lax.fori_loop(0, _NCHUNK, _step, 0)

    plsc.subcore_barrier()

    pltpu.sync_copy(acc_sh.at[pl.ds(base, _STRIPE)],
                    seg_out.at[cid, pl.ds(base, _STRIPE)])
    if with_cnt:
        pltpu.sync_copy(cnt_sh.at[pl.ds(base, _STRIPE)],
                        cnt_out.at[cid, pl.ds(base, _STRIPE)])


def _make_seg(with_cnt):
    mesh = plsc.VectorSubcoreMesh(core_axis_name="c", subcore_axis_name="s")
    out_type = [jax.ShapeDtypeStruct((_NC, N_NODES, D_FEAT), jnp.float32)]
    scratch = [pltpu.VMEM_SHARED((N_NODES, D_FEAT), jnp.float32)]
    if with_cnt:
        out_type.append(jax.ShapeDtypeStruct((_NC, N_NODES, _CNTW),
                                             jnp.float32))
        scratch.append(pltpu.VMEM_SHARED((N_NODES, _CNTW), jnp.float32))
    scratch += [
        pltpu.VMEM((_NCHUNK, _CHUNK), jnp.int32),
        pltpu.VMEM((_NCHUNK, _CHUNK), jnp.int32),
        pltpu.VMEM((_CHUNK, D_FEAT), jnp.float32),
    ]
    if with_cnt:
        scratch.append(pltpu.VMEM((_CHUNK, _CNTW), jnp.float32))
    scratch.append(pltpu.VMEM((_ZROWS, D_FEAT), jnp.float32))
    if with_cnt:
        scratch.append(pltpu.VMEM((_ZROWS, _CNTW), jnp.float32))
    scratch.append(pltpu.SemaphoreType.DMA)
    return pl.kernel(functools.partial(_seg_body, with_cnt),
                     out_type=out_type, mesh=mesh, scratch_types=scratch)


def _lin2_body(x_ref, wl_ref, wr_ref, b_ref, y_ref, r_ref):
    xb = x_ref[...]
    y_ref[...] = jnp.dot(xb, wl_ref[...], preferred_element_type=jnp.float32)
    r_ref[...] = (jnp.dot(xb, wr_ref[...], preferred_element_type=jnp.float32)
                  + b_ref[...])


def _mid_body(seg_ref, cnt_ref, r1_ref, wl_ref, wr_ref, b_ref, y_ref, r_ref):
    seg = seg_ref[0] + seg_ref[1]
    c = cnt_ref[0, :, 0:1] + cnt_ref[1, :, 0:1]
    inv = 1.0 / jnp.maximum(c, 1.0)
    h = jnp.maximum(seg * inv + r1_ref[...], 0.0)
    y_ref[...] = jnp.dot(h, wl_ref[...], preferred_element_type=jnp.float32)
    r_ref[...] = (jnp.dot(h, wr_ref[...], preferred_element_type=jnp.float32)
                  + b_ref[...])


def _fin_body(seg_ref, cnt_ref, r2_ref, out_ref):
    seg = seg_ref[0] + seg_ref[1]
    c = cnt_ref[0, :, 0:1] + cnt_ref[1, :, 0:1]
    inv = 1.0 / jnp.maximum(c, 1.0)
    out_ref[...] = seg * inv + r2_ref[...]


_rowspec = pl.BlockSpec((_ROWBLK, D_FEAT), lambda i: (i, 0))
_wspec = pl.BlockSpec((D_FEAT, HIDDEN), lambda i: (0, 0))
_bspec = pl.BlockSpec((1, HIDDEN), lambda i: (0, 0))
_segspec = pl.BlockSpec((_NC, _ROWBLK, D_FEAT), lambda i: (0, i, 0))
_cntspec = pl.BlockSpec((_NC, _ROWBLK, _CNTW), lambda i: (0, i, 0))
_out2 = [jax.ShapeDtypeStruct((N_NODES, HIDDEN), jnp.float32)] * 2

_lin2 = pl.pallas_call(
    _lin2_body, grid=(_NBLK,),
    in_specs=[_rowspec, _wspec, _wspec, _bspec],
    out_specs=[_rowspec, _rowspec],
    out_shape=_out2,
)

_mid = pl.pallas_call(
    _mid_body, grid=(_NBLK,),
    in_specs=[_segspec, _cntspec, _rowspec, _wspec, _wspec, _bspec],
    out_specs=[_rowspec, _rowspec],
    out_shape=_out2,
)

_fin = pl.pallas_call(
    _fin_body, grid=(_NBLK,),
    in_specs=[_segspec, _cntspec, _rowspec],
    out_specs=_rowspec,
    out_shape=jax.ShapeDtypeStruct((N_NODES, HIDDEN), jnp.float32),
)

_seg_cnt = _make_seg(True)
_seg_only = _make_seg(False)


def kernel(x, edge_index, W1_l, W1_r, b1, W2_l, W2_r, b2):
    src = edge_index[0].astype(jnp.int32).reshape(_NW, _NCHUNK, _CHUNK)
    dst = edge_index[1].astype(jnp.int32).reshape(_NW, _NCHUNK, _CHUNK)
    b1r = b1.reshape(1, HIDDEN)
    b2r = b2.reshape(1, HIDDEN)

    y1, r1 = _lin2(x, W1_l, W1_r, b1r)
    seg1, cnt = _seg_cnt(y1, src, dst)
    y2, r2 = _mid(seg1, cnt, r1, W2_l, W2_r, b2r)
    (seg2,) = _seg_only(y2, src, dst)
    out = _fin(seg2, cnt, r2)
    return out


# speedup vs baseline: 6.9486x; 6.9486x over previous
import functools

import jax
import jax.numpy as jnp
from jax import lax
from jax.experimental import pallas as pl
from jax.experimental.pallas import tpu as pltpu
from jax.experimental.pallas import tpu_sc as plsc

N_NODES = 10000
D_FEAT = 128
HIDDEN = 128
N_EDGES = 320000

_NC = 2
_NS = 16
_NW = _NC * _NS
_EPW = N_EDGES // _NW
_CHUNK = 80
_NCHUNK = _EPW // _CHUNK
_CPB = 25
_KBLK = _NCHUNK // _CPB
_NPAD = 10240
_STRIPE = _NPAD // _NS
_ZROWS = 64
_CNTW = 16

_ROWBLK = 400
_NBLK = N_NODES // _ROWBLK


def _fill_idx(idx_v, start):
    for i in range(_CHUNK // 16):
        idx_v[pl.ds(i * 16, 16)] = lax.iota(jnp.int32, 16) + start + i * 16


def _seg_body(with_cnt, y_hbm, src_hbm, dst_hbm, seg_out, *rest):
    if with_cnt:
        (cnt_out, acc_sh, cnt_sh, src_v, dst_v, idx_v, sidx_v, rows_v,
         ones_v, i128_v, cbuf_v, sem) = rest
    else:
        acc_sh, src_v, dst_v, idx_v, sidx_v, rows_v, sem = rest
    cid = lax.axis_index("c")
    sid = lax.axis_index("s")
    wid = cid * _NS + sid
    base = sid * _STRIPE

    def _zb(i, c):
        rows_v[i // 8, pl.ds((i % 8) * 16, 16)] = jnp.zeros((16,),
                                                            jnp.float32)
        return c
    lax.fori_loop(0, _CHUNK * 8, _zb, 0)
    if with_cnt:
        for i in range(_CHUNK // 16):
            ones_v[pl.ds(i * 16, 16)] = jnp.zeros((16,), jnp.float32)
    for k in range(_STRIPE // _CHUNK):
        _fill_idx(idx_v, base + k * _CHUNK)
        pltpu.sync_copy(rows_v, acc_sh.at[idx_v])
        if with_cnt:
            pltpu.sync_copy(ones_v, cnt_sh.at[idx_v])
    if with_cnt:
        for i in range(_CHUNK // 16):
            ones_v[pl.ds(i * 16, 16)] = jnp.ones((16,), jnp.float32)

    plsc.subcore_barrier()

    for kb in range(_KBLK):
        pltpu.sync_copy(src_hbm.at[wid, kb], src_v)
        pltpu.sync_copy(dst_hbm.at[wid, kb], dst_v)

        def _step(j, c2):
            for i in range(_CHUNK // 16):
                idx_v[pl.ds(i * 16, 16)] = dst_v[j, pl.ds(i * 16, 16)]
                sidx_v[pl.ds(i * 16, 16)] = src_v[j, pl.ds(i * 16, 16)]
            pltpu.async_copy(y_hbm.at[sidx_v], rows_v, sem).wait()
            pltpu.sync_copy(rows_v, acc_sh.at[idx_v], add=True)
            if with_cnt:
                pltpu.sync_copy(ones_v, cnt_sh.at[idx_v], add=True)
            return c2
        lax.fori_loop(0, _CPB, _step, 0)

    plsc.subcore_barrier()

    for k in range(_STRIPE // _CHUNK):
        row = base + k * _CHUNK
        _fill_idx(idx_v, row)
        pltpu.async_copy(acc_sh.at[idx_v], rows_v, sem).wait()
        pltpu.sync_copy(rows_v, seg_out.at[cid, pl.ds(row, _CHUNK)])
    if with_cnt:
        for k in range(_STRIPE // 128):
            r = sid * (_STRIPE // 128) + k
            for i in range(8):
                i128_v[pl.ds(i * 16, 16)] = (lax.iota(jnp.int32, 16)
                                             + r * 128 + i * 16)
            pltpu.async_copy(cnt_sh.at[i128_v], cbuf_v, sem).wait()
            pltpu.sync_copy(cbuf_v, cnt_out.at[cid, r])


@functools.lru_cache(maxsize=None)
def _make_seg(with_cnt):
    mesh = plsc.VectorSubcoreMesh(core_axis_name="c", subcore_axis_name="s")
    out_type = [jax.ShapeDtypeStruct((_NC, _NPAD, D_FEAT), jnp.float32)]
    scratch = [pltpu.VMEM_SHARED((_NPAD, D_FEAT), jnp.float32)]
    if with_cnt:
        out_type.append(jax.ShapeDtypeStruct((_NC, _NPAD // 128, 128),
                                             jnp.float32))
        scratch.append(pltpu.VMEM_SHARED((_NPAD,), jnp.float32))
    scratch += [
        pltpu.VMEM((_CPB, _CHUNK), jnp.int32),
        pltpu.VMEM((_CPB, _CHUNK), jnp.int32),
        pltpu.VMEM((_CHUNK,), jnp.int32),
        pltpu.VMEM((_CHUNK,), jnp.int32),
        pltpu.VMEM((_CHUNK, D_FEAT), jnp.float32),
    ]
    if with_cnt:
        scratch += [
            pltpu.VMEM((_CHUNK,), jnp.float32),
            pltpu.VMEM((128,), jnp.int32),
            pltpu.VMEM((128,), jnp.float32),
        ]
    scratch.append(pltpu.SemaphoreType.DMA)
    return pl.kernel(functools.partial(_seg_body, with_cnt),
                     out_type=out_type, mesh=mesh, scratch_types=scratch)


def _lin2_body(x_ref, wl_ref, wr_ref, b_ref, y_ref, r_ref):
    xb = x_ref[...]
    y_ref[...] = jnp.dot(xb, wl_ref[...], preferred_element_type=jnp.float32)
    r_ref[...] = (jnp.dot(xb, wr_ref[...], preferred_element_type=jnp.float32)
                  + b_ref[...])


def _mid_body(seg_ref, cnt_ref, r1_ref, wl_ref, wr_ref, b_ref, y_ref, r_ref):
    seg = seg_ref[0] + seg_ref[1]
    c = cnt_ref[0] + cnt_ref[1]
    inv = 1.0 / jnp.maximum(c, 1.0)
    h = jnp.maximum(seg * inv + r1_ref[...], 0.0)
    y_ref[...] = jnp.dot(h, wl_ref[...], preferred_element_type=jnp.float32)
    r_ref[...] = (jnp.dot(h, wr_ref[...], preferred_element_type=jnp.float32)
                  + b_ref[...])


def _fin_body(seg_ref, cnt_ref, r2_ref, out_ref):
    seg = seg_ref[0] + seg_ref[1]
    c = cnt_ref[0] + cnt_ref[1]
    inv = 1.0 / jnp.maximum(c, 1.0)
    out_ref[...] = seg * inv + r2_ref[...]


_rowspec = pl.BlockSpec((_ROWBLK, D_FEAT), lambda i: (i, 0))
_wspec = pl.BlockSpec((D_FEAT, HIDDEN), lambda i: (0, 0))
_bspec = pl.BlockSpec((1, HIDDEN), lambda i: (0, 0))
_segspec = pl.BlockSpec((_NC, _ROWBLK, D_FEAT), lambda i: (0, i, 0))
_cntspec = pl.BlockSpec((_NC, _ROWBLK, 1), lambda i: (0, i, 0))
_out2 = [jax.ShapeDtypeStruct((N_NODES, HIDDEN), jnp.float32)] * 2

_lin2 = pl.pallas_call(
    _lin2_body, grid=(_NBLK,),
    in_specs=[_rowspec, _wspec, _wspec, _bspec],
    out_specs=[_rowspec, _rowspec],
    out_shape=_out2,
)

_mid = pl.pallas_call(
    _mid_body, grid=(_NBLK,),
    in_specs=[_segspec, _cntspec, _rowspec, _wspec, _wspec, _bspec],
    out_specs=[_rowspec, _rowspec],
    out_shape=_out2,
)

_fin = pl.pallas_call(
    _fin_body, grid=(_NBLK,),
    in_specs=[_segspec, _cntspec, _rowspec],
    out_specs=_rowspec,
    out_shape=jax.ShapeDtypeStruct((N_NODES, HIDDEN), jnp.float32),
)


def kernel(x, edge_index, W1_l, W1_r, b1, W2_l, W2_r, b2):
    src = edge_index[0].astype(jnp.int32).reshape(_NW, _KBLK, _CPB, _CHUNK)
    dst = edge_index[1].astype(jnp.int32).reshape(_NW, _KBLK, _CPB, _CHUNK)
    b1r = b1.reshape(1, HIDDEN)
    b2r = b2.reshape(1, HIDDEN)

    y1, r1 = _lin2(x, W1_l, W1_r, b1r)
    seg1, cntp = _make_seg(True)(y1, src, dst)
    cnt = cntp.reshape(_NC, _NPAD, 1)
    y2, r2 = _mid(seg1, cnt, r1, W2_l, W2_r, b2r)
    (seg2,) = _make_seg(False)(y2, src, dst)
    out = _fin(seg2, cnt, r2)
    return out


# speedup vs baseline: 8.0216x; 1.1544x over previous
import functools

import jax
import jax.numpy as jnp
from jax import lax
from jax.experimental import pallas as pl
from jax.experimental.pallas import tpu as pltpu
from jax.experimental.pallas import tpu_sc as plsc

N_NODES = 10000
D_FEAT = 128
HIDDEN = 128
N_EDGES = 320000

_NC = 2
_NS = 16
_NW = _NC * _NS
_EPW = N_EDGES // _NW
_CHUNK = 80
_NCHUNK = _EPW // _CHUNK
_CPB = 25
_KBLK = _NCHUNK // _CPB
_NPAD = 10240
_STRIPE = _NPAD // _NS
_ZROWS = 64
_CNTW = 16

_ROWBLK = 400
_NBLK = N_NODES // _ROWBLK


def _fill_idx(idx_v, start):
    for i in range(_CHUNK // 16):
        idx_v[pl.ds(i * 16, 16)] = lax.iota(jnp.int32, 16) + start + i * 16


def _seg_body(with_cnt, y_hbm, src_hbm, dst_hbm, seg_out, *rest):
    if with_cnt:
        (cnt_out, acc_sh, cnt_sh, idx0, sidx0, idx1, sidx1, rows0, rows1,
         ones_v, i128_v, cbuf_v, gsem0, gsem1) = rest
    else:
        (acc_sh, idx0, sidx0, idx1, sidx1, rows0, rows1,
         gsem0, gsem1) = rest
    cid = lax.axis_index("c")
    sid = lax.axis_index("s")
    wid = cid * _NS + sid
    base = sid * _STRIPE
    woff = wid * _EPW
    slots = ((idx0, sidx0, rows0, gsem0), (idx1, sidx1, rows1, gsem1))

    def _zb(i, c):
        rows0[i // 8, pl.ds((i % 8) * 16, 16)] = jnp.zeros((16,),
                                                           jnp.float32)
        return c
    lax.fori_loop(0, _CHUNK * 8, _zb, 0)
    if with_cnt:
        for i in range(_CHUNK // 16):
            ones_v[pl.ds(i * 16, 16)] = jnp.zeros((16,), jnp.float32)
    for k in range(_STRIPE // _CHUNK):
        _fill_idx(idx0, base + k * _CHUNK)
        pltpu.sync_copy(rows0, acc_sh.at[idx0])
        if with_cnt:
            pltpu.sync_copy(ones_v, cnt_sh.at[idx0])
    if with_cnt:
        for i in range(_CHUNK // 16):
            ones_v[pl.ds(i * 16, 16)] = jnp.ones((16,), jnp.float32)

    plsc.subcore_barrier()

    def _ld(j, idxb, sidxb):
        pltpu.sync_copy(dst_hbm.at[pl.ds(woff + j * _CHUNK, _CHUNK)], idxb)
        pltpu.sync_copy(src_hbm.at[pl.ds(woff + j * _CHUNK, _CHUNK)], sidxb)

    def _scat(idxb, rowsb):
        pltpu.sync_copy(rowsb, acc_sh.at[idxb], add=True)
        if with_cnt:
            pltpu.sync_copy(ones_v, cnt_sh.at[idxb], add=True)

    _ld(0, idx0, sidx0)
    pltpu.async_copy(y_hbm.at[sidx0], rows0, gsem0)
    _ld(1, idx1, sidx1)

    def _body(g, c):
        j = 2 * g
        pltpu.async_copy(y_hbm.at[sidx1], rows1, gsem1)
        pltpu.make_async_copy(y_hbm.at[sidx0], rows0, gsem0).wait()
        _scat(idx0, rows0)
        _ld(j + 2, idx0, sidx0)
        pltpu.async_copy(y_hbm.at[sidx0], rows0, gsem0)
        pltpu.make_async_copy(y_hbm.at[sidx1], rows1, gsem1).wait()
        _scat(idx1, rows1)
        _ld(j + 3, idx1, sidx1)
        return c
    lax.fori_loop(0, (_NCHUNK - 1) // 2, _body, 0)
    pltpu.make_async_copy(y_hbm.at[sidx0], rows0, gsem0).wait()
    _scat(idx0, rows0)

    plsc.subcore_barrier()

    _fill_idx(idx0, base)
    pltpu.async_copy(acc_sh.at[idx0], rows0, gsem0)
    for k in range(_STRIPE // _CHUNK):
        idxb, _, rowsb, semb = slots[k % 2]
        if k + 1 < _STRIPE // _CHUNK:
            idxn, _, rowsn, semn = slots[(k + 1) % 2]
            _fill_idx(idxn, base + (k + 1) * _CHUNK)
            pltpu.async_copy(acc_sh.at[idxn], rowsn, semn)
        pltpu.make_async_copy(acc_sh.at[idxb], rowsb, semb).wait()
        pltpu.sync_copy(rowsb, seg_out.at[cid, pl.ds(base + k * _CHUNK,
                                                     _CHUNK)])
    if with_cnt:
        for k in range(_STRIPE // 128):
            r = sid * (_STRIPE // 128) + k
            for i in range(8):
                i128_v[pl.ds(i * 16, 16)] = (lax.iota(jnp.int32, 16)
                                             + r * 128 + i * 16)
            pltpu.async_copy(cnt_sh.at[i128_v], cbuf_v, gsem0).wait()
            pltpu.sync_copy(cbuf_v, cnt_out.at[cid, r])


@functools.lru_cache(maxsize=None)
def _make_seg(with_cnt):
    mesh = plsc.VectorSubcoreMesh(core_axis_name="c", subcore_axis_name="s")
    out_type = [jax.ShapeDtypeStruct((_NC, _NPAD, D_FEAT), jnp.float32)]
    scratch = [pltpu.VMEM_SHARED((_NPAD, D_FEAT), jnp.float32)]
    if with_cnt:
        out_type.append(jax.ShapeDtypeStruct((_NC, _NPAD // 128, 128),
                                             jnp.float32))
        scratch.append(pltpu.VMEM_SHARED((_NPAD,), jnp.float32))
    scratch += [
        pltpu.VMEM((_CHUNK,), jnp.int32),
        pltpu.VMEM((_CHUNK,), jnp.int32),
        pltpu.VMEM((_CHUNK,), jnp.int32),
        pltpu.VMEM((_CHUNK,), jnp.int32),
        pltpu.VMEM((_CHUNK, D_FEAT), jnp.float32),
        pltpu.VMEM((_CHUNK, D_FEAT), jnp.float32),
    ]
    if with_cnt:
        scratch += [
            pltpu.VMEM((_CHUNK,), jnp.float32),
            pltpu.VMEM((128,), jnp.int32),
            pltpu.VMEM((128,), jnp.float32),
        ]
    scratch += [pltpu.SemaphoreType.DMA, pltpu.SemaphoreType.DMA]
    return pl.kernel(functools.partial(_seg_body, with_cnt),
                     out_type=out_type, mesh=mesh, scratch_types=scratch)


def _lin2_body(x_ref, wl_ref, wr_ref, b_ref, y_ref, r_ref):
    xb = x_ref[...]
    y_ref[...] = jnp.dot(xb, wl_ref[...], preferred_element_type=jnp.float32)
    r_ref[...] = (jnp.dot(xb, wr_ref[...], preferred_element_type=jnp.float32)
                  + b_ref[...])


def _mid_body(seg_ref, cnt_ref, r1_ref, wl_ref, wr_ref, b_ref, y_ref, r_ref):
    seg = seg_ref[0] + seg_ref[1]
    c = cnt_ref[0] + cnt_ref[1]
    inv = 1.0 / jnp.maximum(c, 1.0)
    h = jnp.maximum(seg * inv + r1_ref[...], 0.0)
    y_ref[...] = jnp.dot(h, wl_ref[...], preferred_element_type=jnp.float32)
    r_ref[...] = (jnp.dot(h, wr_ref[...], preferred_element_type=jnp.float32)
                  + b_ref[...])


def _fin_body(seg_ref, cnt_ref, r2_ref, out_ref):
    seg = seg_ref[0] + seg_ref[1]
    c = cnt_ref[0] + cnt_ref[1]
    inv = 1.0 / jnp.maximum(c, 1.0)
    out_ref[...] = seg * inv + r2_ref[...]


_rowspec = pl.BlockSpec((_ROWBLK, D_FEAT), lambda i: (i, 0))
_wspec = pl.BlockSpec((D_FEAT, HIDDEN), lambda i: (0, 0))
_bspec = pl.BlockSpec((1, HIDDEN), lambda i: (0, 0))
_segspec = pl.BlockSpec((_NC, _ROWBLK, D_FEAT), lambda i: (0, i, 0))
_cntspec = pl.BlockSpec((_NC, _ROWBLK, 1), lambda i: (0, i, 0))
_out2 = [jax.ShapeDtypeStruct((N_NODES, HIDDEN), jnp.float32)] * 2

_lin2 = pl.pallas_call(
    _lin2_body, grid=(_NBLK,),
    in_specs=[_rowspec, _wspec, _wspec, _bspec],
    out_specs=[_rowspec, _rowspec],
    out_shape=_out2,
)

_mid = pl.pallas_call(
    _mid_body, grid=(_NBLK,),
    in_specs=[_segspec, _cntspec, _rowspec, _wspec, _wspec, _bspec],
    out_specs=[_rowspec, _rowspec],
    out_shape=_out2,
)

_fin = pl.pallas_call(
    _fin_body, grid=(_NBLK,),
    in_specs=[_segspec, _cntspec, _rowspec],
    out_specs=_rowspec,
    out_shape=jax.ShapeDtypeStruct((N_NODES, HIDDEN), jnp.float32),
)


def kernel(x, edge_index, W1_l, W1_r, b1, W2_l, W2_r, b2):
    src = jnp.pad(edge_index[0].astype(jnp.int32), (0, _CHUNK))
    dst = jnp.pad(edge_index[1].astype(jnp.int32), (0, _CHUNK))
    b1r = b1.reshape(1, HIDDEN)
    b2r = b2.reshape(1, HIDDEN)

    y1, r1 = _lin2(x, W1_l, W1_r, b1r)
    seg1, cntp = _make_seg(True)(y1, src, dst)
    cnt = cntp.reshape(_NC, _NPAD, 1)
    y2, r2 = _mid(seg1, cnt, r1, W2_l, W2_r, b2r)
    (seg2,) = _make_seg(False)(y2, src, dst)
    out = _fin(seg2, cnt, r2)
    return out


# speedup vs baseline: 9.2649x; 1.1550x over previous
import functools

import jax
import jax.numpy as jnp
from jax import lax
from jax.experimental import pallas as pl
from jax.experimental.pallas import tpu as pltpu
from jax.experimental.pallas import tpu_sc as plsc

N_NODES = 10000
D_FEAT = 128
HIDDEN = 128
N_EDGES = 320000

_NC = 2
_NS = 16
_NW = _NC * _NS
_EPW = N_EDGES // _NW
_CHUNK = 80
_NCHUNK = _EPW // _CHUNK
_CPB = 25
_KBLK = _NCHUNK // _CPB
_NPAD = 10240
_STRIPE = _NPAD // _NS
_ZROWS = 64
_CNTW = 16

_ROWBLK = 400
_NBLK = N_NODES // _ROWBLK


def _fill_idx(idx_v, start):
    for i in range(_CHUNK // 16):
        idx_v[pl.ds(i * 16, 16)] = lax.iota(jnp.int32, 16) + start + i * 16


_DEPTH = 4


def _seg_body(with_cnt, y_hbm, src_hbm, dst_hbm, seg_out, *rest):
    if with_cnt:
        (cnt_out, acc_sh, cnt_sh, dstg_v, srcg_v, didx, sidx, rows, gsem,
         ssem, ones_v, csem, i128_v, cbuf_v) = rest
    else:
        (acc_sh, dstg_v, srcg_v, didx, sidx, rows, gsem, ssem) = rest
    cid = lax.axis_index("c")
    sid = lax.axis_index("s")
    wid = cid * _NS + sid
    base = sid * _STRIPE
    woff = wid * _EPW

    def _zb(i, c):
        rows[0][i // 8, pl.ds((i % 8) * 16, 16)] = jnp.zeros((16,),
                                                             jnp.float32)
        return c
    lax.fori_loop(0, _CHUNK * 8, _zb, 0)
    if with_cnt:
        for i in range(_CHUNK // 16):
            ones_v[pl.ds(i * 16, 16)] = jnp.zeros((16,), jnp.float32)
    for k in range(_STRIPE // _CHUNK):
        _fill_idx(didx[0], base + k * _CHUNK)
        pltpu.sync_copy(rows[0], acc_sh.at[didx[0]])
        if with_cnt:
            pltpu.sync_copy(ones_v, cnt_sh.at[didx[0]])
    if with_cnt:
        for i in range(_CHUNK // 16):
            ones_v[pl.ds(i * 16, 16)] = jnp.ones((16,), jnp.float32)

    plsc.subcore_barrier()

    blk = _DEPTH * _CHUNK

    def _stage(g):
        pltpu.sync_copy(dst_hbm.at[pl.ds(woff + g * blk, blk)], dstg_v)
        pltpu.sync_copy(src_hbm.at[pl.ds(woff + g * blk, blk)], srcg_v)

    def _unpack(u):
        for i in range(_CHUNK // 16):
            didx[u][pl.ds(i * 16, 16)] = dstg_v[pl.ds(u * _CHUNK + i * 16,
                                                      16)]
            sidx[u][pl.ds(i * 16, 16)] = srcg_v[pl.ds(u * _CHUNK + i * 16,
                                                      16)]

    _stage(0)

    def _body(g, c):
        for u in range(_DEPTH):
            _unpack(u)
            pltpu.async_copy(y_hbm.at[sidx[u]], rows[u], gsem[u])
        _stage(g + 1)
        for u in range(_DEPTH):
            pltpu.make_async_copy(y_hbm.at[sidx[u]], rows[u], gsem[u]).wait()
            pltpu.async_copy(rows[u], acc_sh.at[didx[u]], ssem[u], add=True)
            if with_cnt:
                pltpu.async_copy(ones_v, cnt_sh.at[didx[u]], csem[u],
                                 add=True)
        for u in range(_DEPTH):
            pltpu.make_async_copy(rows[u], acc_sh.at[didx[u]],
                                  ssem[u]).wait()
            if with_cnt:
                pltpu.make_async_copy(ones_v, cnt_sh.at[didx[u]],
                                      csem[u]).wait()
        return c
    lax.fori_loop(0, _NCHUNK // _DEPTH, _body, 0)

    for u in range(_NCHUNK % _DEPTH):
        _unpack(u)
        pltpu.async_copy(y_hbm.at[sidx[u]], rows[u], gsem[u])
    for u in range(_NCHUNK % _DEPTH):
        pltpu.make_async_copy(y_hbm.at[sidx[u]], rows[u], gsem[u]).wait()
        pltpu.sync_copy(rows[u], acc_sh.at[didx[u]], add=True)
        if with_cnt:
            pltpu.sync_copy(ones_v, cnt_sh.at[didx[u]], add=True)

    plsc.subcore_barrier()

    nw = _STRIPE // _CHUNK
    for k in range(nw):
        _fill_idx(didx[k % _DEPTH], base + k * _CHUNK)
        pltpu.async_copy(acc_sh.at[didx[k % _DEPTH]], rows[k % _DEPTH],
                         gsem[k % _DEPTH])
        if k % _DEPTH == _DEPTH - 1 or k == nw - 1:
            for kk in range(k - k % _DEPTH, k + 1):
                u = kk % _DEPTH
                pltpu.make_async_copy(acc_sh.at[didx[u]], rows[u],
                                      gsem[u]).wait()
                pltpu.sync_copy(rows[u],
                                seg_out.at[cid, pl.ds(base + kk * _CHUNK,
                                                      _CHUNK)])
    if with_cnt:
        for k in range(_STRIPE // 128):
            r = sid * (_STRIPE // 128) + k
            for i in range(8):
                i128_v[pl.ds(i * 16, 16)] = (lax.iota(jnp.int32, 16)
                                             + r * 128 + i * 16)
            pltpu.async_copy(cnt_sh.at[i128_v], cbuf_v, gsem[0]).wait()
            pltpu.sync_copy(cbuf_v, cnt_out.at[cid, r])


@functools.lru_cache(maxsize=None)
def _make_seg(with_cnt):
    mesh = plsc.VectorSubcoreMesh(core_axis_name="c", subcore_axis_name="s")
    out_type = [jax.ShapeDtypeStruct((_NC, _NPAD, D_FEAT), jnp.float32)]
    scratch = [pltpu.VMEM_SHARED((_NPAD, D_FEAT), jnp.float32)]
    if with_cnt:
        out_type.append(jax.ShapeDtypeStruct((_NC, _NPAD // 128, 128),
                                             jnp.float32))
        scratch.append(pltpu.VMEM_SHARED((_NPAD,), jnp.float32))
    scratch += [
        pltpu.VMEM((_DEPTH * _CHUNK,), jnp.int32),
        pltpu.VMEM((_DEPTH * _CHUNK,), jnp.int32),
        [pltpu.VMEM((_CHUNK,), jnp.int32)] * _DEPTH,
        [pltpu.VMEM((_CHUNK,), jnp.int32)] * _DEPTH,
        [pltpu.VMEM((_CHUNK, D_FEAT), jnp.float32)] * _DEPTH,
        [pltpu.SemaphoreType.DMA] * _DEPTH,
        [pltpu.SemaphoreType.DMA] * _DEPTH,
    ]
    if with_cnt:
        scratch += [
            pltpu.VMEM((_CHUNK,), jnp.float32),
            [pltpu.SemaphoreType.DMA] * _DEPTH,
            pltpu.VMEM((128,), jnp.int32),
            pltpu.VMEM((128,), jnp.float32),
        ]
    return pl.kernel(functools.partial(_seg_body, with_cnt),
                     out_type=out_type, mesh=mesh, scratch_types=scratch)


def _lin2_body(x_ref, wl_ref, wr_ref, b_ref, y_ref, r_ref):
    xb = x_ref[...]
    y_ref[...] = jnp.dot(xb, wl_ref[...], preferred_element_type=jnp.float32)
    r_ref[...] = (jnp.dot(xb, wr_ref[...], preferred_element_type=jnp.float32)
                  + b_ref[...])


def _mid_body(seg_ref, cnt_ref, r1_ref, wl_ref, wr_ref, b_ref, y_ref, r_ref):
    seg = seg_ref[0] + seg_ref[1]
    c = cnt_ref[0] + cnt_ref[1]
    inv = 1.0 / jnp.maximum(c, 1.0)
    h = jnp.maximum(seg * inv + r1_ref[...], 0.0)
    y_ref[...] = jnp.dot(h, wl_ref[...], preferred_element_type=jnp.float32)
    r_ref[...] = (jnp.dot(h, wr_ref[...], preferred_element_type=jnp.float32)
                  + b_ref[...])


def _fin_body(seg_ref, cnt_ref, r2_ref, out_ref):
    seg = seg_ref[0] + seg_ref[1]
    c = cnt_ref[0] + cnt_ref[1]
    inv = 1.0 / jnp.maximum(c, 1.0)
    out_ref[...] = seg * inv + r2_ref[...]


_rowspec = pl.BlockSpec((_ROWBLK, D_FEAT), lambda i: (i, 0))
_wspec = pl.BlockSpec((D_FEAT, HIDDEN), lambda i: (0, 0))
_bspec = pl.BlockSpec((1, HIDDEN), lambda i: (0, 0))
_segspec = pl.BlockSpec((_NC, _ROWBLK, D_FEAT), lambda i: (0, i, 0))
_cntspec = pl.BlockSpec((_NC, _ROWBLK, 1), lambda i: (0, i, 0))
_out2 = [jax.ShapeDtypeStruct((N_NODES, HIDDEN), jnp.float32)] * 2

_lin2 = pl.pallas_call(
    _lin2_body, grid=(_NBLK,),
    in_specs=[_rowspec, _wspec, _wspec, _bspec],
    out_specs=[_rowspec, _rowspec],
    out_shape=_out2,
)

_mid = pl.pallas_call(
    _mid_body, grid=(_NBLK,),
    in_specs=[_segspec, _cntspec, _rowspec, _wspec, _wspec, _bspec],
    out_specs=[_rowspec, _rowspec],
    out_shape=_out2,
)

_fin = pl.pallas_call(
    _fin_body, grid=(_NBLK,),
    in_specs=[_segspec, _cntspec, _rowspec],
    out_specs=_rowspec,
    out_shape=jax.ShapeDtypeStruct((N_NODES, HIDDEN), jnp.float32),
)


def kernel(x, edge_index, W1_l, W1_r, b1, W2_l, W2_r, b2):
    src = jnp.pad(edge_index[0].astype(jnp.int32), (0, _DEPTH * _CHUNK))
    dst = jnp.pad(edge_index[1].astype(jnp.int32), (0, _DEPTH * _CHUNK))
    b1r = b1.reshape(1, HIDDEN)
    b2r = b2.reshape(1, HIDDEN)

    y1, r1 = _lin2(x, W1_l, W1_r, b1r)
    seg1, cntp = _make_seg(True)(y1, src, dst)
    cnt = cntp.reshape(_NC, _NPAD, 1)
    y2, r2 = _mid(seg1, cnt, r1, W2_l, W2_r, b2r)
    (seg2,) = _make_seg(False)(y2, src, dst)
    out = _fin(seg2, cnt, r2)
    return out


# speedup vs baseline: 11.6539x; 1.2578x over previous
import functools

import jax
import jax.numpy as jnp
from jax import lax
from jax.experimental import pallas as pl
from jax.experimental.pallas import tpu as pltpu
from jax.experimental.pallas import tpu_sc as plsc

N_NODES = 10000
D_FEAT = 128
HIDDEN = 128
N_EDGES = 320000

_NC = 2
_NS = 16
_NW = _NC * _NS
_EPW = N_EDGES // _NW
_CHUNK = 80
_NCHUNK = _EPW // _CHUNK
_CPB = 25
_KBLK = _NCHUNK // _CPB
_NPAD = 10240
_STRIPE = _NPAD // _NS
_ZROWS = 64
_CNTW = 16

_ROWBLK = 400
_NBLK = N_NODES // _ROWBLK


def _fill_idx(idx_v, start):
    for i in range(_CHUNK // 16):
        idx_v[pl.ds(i * 16, 16)] = lax.iota(jnp.int32, 16) + start + i * 16


_DEPTH = 4


def _seg_body(with_cnt, y_hbm, src_hbm, dst_hbm, seg_out, *rest):
    if with_cnt:
        (cnt_out, acc_sh, cnt_sh, stg0, stg1, isem, didx, sidx, rows, gsem,
         ssem, ones_v, csem, i128_v, cbuf_v) = rest
    else:
        (acc_sh, stg0, stg1, isem, didx, sidx, rows, gsem, ssem) = rest
    cid = lax.axis_index("c")
    sid = lax.axis_index("s")
    wid = cid * _NS + sid
    base = sid * _STRIPE
    woff = wid * _EPW

    def _zb(i, c):
        rows[0][i // 8, pl.ds((i % 8) * 16, 16)] = jnp.zeros((16,),
                                                             jnp.float32)
        return c
    lax.fori_loop(0, _CHUNK * 8, _zb, 0)
    if with_cnt:
        for i in range(_CHUNK // 16):
            ones_v[pl.ds(i * 16, 16)] = jnp.zeros((16,), jnp.float32)
    for k in range(_STRIPE // _CHUNK):
        _fill_idx(didx[0], base + k * _CHUNK)
        pltpu.sync_copy(rows[0], acc_sh.at[didx[0]])
        if with_cnt:
            pltpu.sync_copy(ones_v, cnt_sh.at[didx[0]])
    if with_cnt:
        for i in range(_CHUNK // 16):
            ones_v[pl.ds(i * 16, 16)] = jnp.ones((16,), jnp.float32)

    plsc.subcore_barrier()

    blk = _DEPTH * _CHUNK
    stg = (stg0, stg1)

    def _stage(g, p):
        pltpu.async_copy(dst_hbm.at[pl.ds(woff + g * blk, blk)],
                         stg[p][0], isem[p])
        pltpu.async_copy(src_hbm.at[pl.ds(woff + g * blk, blk)],
                         stg[p][1], isem[p])

    def _stage_wait(g, p):
        pltpu.make_async_copy(dst_hbm.at[pl.ds(woff + g * blk, blk)],
                              stg[p][0], isem[p]).wait()
        pltpu.make_async_copy(src_hbm.at[pl.ds(woff + g * blk, blk)],
                              stg[p][1], isem[p]).wait()

    def _unpack(p, u):
        for i in range(_CHUNK // 16):
            didx[u][pl.ds(i * 16, 16)] = stg[p][0][pl.ds(u * _CHUNK
                                                         + i * 16, 16)]
            sidx[u][pl.ds(i * 16, 16)] = stg[p][1][pl.ds(u * _CHUNK
                                                         + i * 16, 16)]

    def _gather(u):
        pltpu.async_copy(y_hbm.at[sidx[u]], rows[u], gsem[u])

    def _gather_wait(u):
        pltpu.make_async_copy(y_hbm.at[sidx[u]], rows[u], gsem[u]).wait()

    def _scat(u):
        pltpu.async_copy(rows[u], acc_sh.at[didx[u]], ssem[u], add=True)
        if with_cnt:
            pltpu.async_copy(ones_v, cnt_sh.at[didx[u]], csem[u], add=True)

    def _scat_wait(u):
        pltpu.make_async_copy(rows[u], acc_sh.at[didx[u]], ssem[u]).wait()
        if with_cnt:
            pltpu.make_async_copy(ones_v, cnt_sh.at[didx[u]], csem[u]).wait()

    _stage(0, 0)
    _stage_wait(0, 0)
    for u in range(_DEPTH):
        _unpack(0, u)
        _gather(u)
    _stage(1, 1)
    for u in range(_DEPTH):
        _gather_wait(u)
        _scat(u)

    def _block(g, p):
        _stage_wait(g, p)
        for u in range(_DEPTH):
            _scat_wait(u)
            _unpack(p, u)
            _gather(u)
        _stage(g + 1, 1 - p)
        for u in range(_DEPTH):
            _gather_wait(u)
            _scat(u)

    def _body(h, c):
        g = 2 * h + 1
        _block(g, 1)
        _block(g + 1, 0)
        return c
    lax.fori_loop(0, (_NCHUNK // _DEPTH - 1) // 2, _body, 0)

    gl = _NCHUNK // _DEPTH
    pl_ = gl % 2
    _stage_wait(gl, pl_)
    for u in range(_NCHUNK % _DEPTH):
        _scat_wait(u)
        _unpack(pl_, u)
        _gather(u)
    for u in range(_NCHUNK % _DEPTH, _DEPTH):
        _scat_wait(u)
    for u in range(_NCHUNK % _DEPTH):
        _gather_wait(u)
        _scat(u)
        _scat_wait(u)

    plsc.subcore_barrier()

    nw = _STRIPE // _CHUNK
    for k in range(nw):
        _fill_idx(didx[k % _DEPTH], base + k * _CHUNK)
        pltpu.async_copy(acc_sh.at[didx[k % _DEPTH]], rows[k % _DEPTH],
                         gsem[k % _DEPTH])
        if k % _DEPTH == _DEPTH - 1 or k == nw - 1:
            for kk in range(k - k % _DEPTH, k + 1):
                u = kk % _DEPTH
                pltpu.make_async_copy(acc_sh.at[didx[u]], rows[u],
                                      gsem[u]).wait()
                pltpu.sync_copy(rows[u],
                                seg_out.at[cid, pl.ds(base + kk * _CHUNK,
                                                      _CHUNK)])
    if with_cnt:
        for k in range(_STRIPE // 128):
            r = sid * (_STRIPE // 128) + k
            for i in range(8):
                i128_v[pl.ds(i * 16, 16)] = (lax.iota(jnp.int32, 16)
                                             + r * 128 + i * 16)
            pltpu.async_copy(cnt_sh.at[i128_v], cbuf_v, gsem[0]).wait()
            pltpu.sync_copy(cbuf_v, cnt_out.at[cid, r])


@functools.lru_cache(maxsize=None)
def _make_seg(with_cnt):
    mesh = plsc.VectorSubcoreMesh(core_axis_name="c", subcore_axis_name="s")
    out_type = [jax.ShapeDtypeStruct((_NC, _NPAD, D_FEAT), jnp.float32)]
    scratch = [pltpu.VMEM_SHARED((_NPAD, D_FEAT), jnp.float32)]
    if with_cnt:
        out_type.append(jax.ShapeDtypeStruct((_NC, _NPAD // 128, 128),
                                             jnp.float32))
        scratch.append(pltpu.VMEM_SHARED((_NPAD,), jnp.float32))
    scratch += [
        [pltpu.VMEM((_DEPTH * _CHUNK,), jnp.int32)] * 2,
        [pltpu.VMEM((_DEPTH * _CHUNK,), jnp.int32)] * 2,
        [pltpu.SemaphoreType.DMA] * 2,
        [pltpu.VMEM((_CHUNK,), jnp.int32)] * _DEPTH,
        [pltpu.VMEM((_CHUNK,), jnp.int32)] * _DEPTH,
        [pltpu.VMEM((_CHUNK, D_FEAT), jnp.float32)] * _DEPTH,
        [pltpu.SemaphoreType.DMA] * _DEPTH,
        [pltpu.SemaphoreType.DMA] * _DEPTH,
    ]
    if with_cnt:
        scratch += [
            pltpu.VMEM((_CHUNK,), jnp.float32),
            [pltpu.SemaphoreType.DMA] * _DEPTH,
            pltpu.VMEM((128,), jnp.int32),
            pltpu.VMEM((128,), jnp.float32),
        ]
    return pl.kernel(functools.partial(_seg_body, with_cnt),
                     out_type=out_type, mesh=mesh, scratch_types=scratch)


def _lin2_body(x_ref, wl_ref, wr_ref, b_ref, y_ref, r_ref):
    xb = x_ref[...]
    y_ref[...] = jnp.dot(xb, wl_ref[...], preferred_element_type=jnp.float32)
    r_ref[...] = (jnp.dot(xb, wr_ref[...], preferred_element_type=jnp.float32)
                  + b_ref[...])


def _mid_body(seg_ref, cnt_ref, r1_ref, wl_ref, wr_ref, b_ref, y_ref, r_ref):
    seg = seg_ref[0] + seg_ref[1]
    c = cnt_ref[0] + cnt_ref[1]
    inv = 1.0 / jnp.maximum(c, 1.0)
    h = jnp.maximum(seg * inv + r1_ref[...], 0.0)
    y_ref[...] = jnp.dot(h, wl_ref[...], preferred_element_type=jnp.float32)
    r_ref[...] = (jnp.dot(h, wr_ref[...], preferred_element_type=jnp.float32)
                  + b_ref[...])


def _fin_body(seg_ref, cnt_ref, r2_ref, out_ref):
    seg = seg_ref[0] + seg_ref[1]
    c = cnt_ref[0] + cnt_ref[1]
    inv = 1.0 / jnp.maximum(c, 1.0)
    out_ref[...] = seg * inv + r2_ref[...]


_rowspec = pl.BlockSpec((_ROWBLK, D_FEAT), lambda i: (i, 0))
_wspec = pl.BlockSpec((D_FEAT, HIDDEN), lambda i: (0, 0))
_bspec = pl.BlockSpec((1, HIDDEN), lambda i: (0, 0))
_segspec = pl.BlockSpec((_NC, _ROWBLK, D_FEAT), lambda i: (0, i, 0))
_cntspec = pl.BlockSpec((_NC, _ROWBLK, 1), lambda i: (0, i, 0))
_out2 = [jax.ShapeDtypeStruct((N_NODES, HIDDEN), jnp.float32)] * 2

_lin2 = pl.pallas_call(
    _lin2_body, grid=(_NBLK,),
    in_specs=[_rowspec, _wspec, _wspec, _bspec],
    out_specs=[_rowspec, _rowspec],
    out_shape=_out2,
)

_mid = pl.pallas_call(
    _mid_body, grid=(_NBLK,),
    in_specs=[_segspec, _cntspec, _rowspec, _wspec, _wspec, _bspec],
    out_specs=[_rowspec, _rowspec],
    out_shape=_out2,
)

_fin = pl.pallas_call(
    _fin_body, grid=(_NBLK,),
    in_specs=[_segspec, _cntspec, _rowspec],
    out_specs=_rowspec,
    out_shape=jax.ShapeDtypeStruct((N_NODES, HIDDEN), jnp.float32),
)


def kernel(x, edge_index, W1_l, W1_r, b1, W2_l, W2_r, b2):
    src = jnp.pad(edge_index[0].astype(jnp.int32), (0, _DEPTH * _CHUNK))
    dst = jnp.pad(edge_index[1].astype(jnp.int32), (0, _DEPTH * _CHUNK))
    b1r = b1.reshape(1, HIDDEN)
    b2r = b2.reshape(1, HIDDEN)

    y1, r1 = _lin2(x, W1_l, W1_r, b1r)
    seg1, cntp = _make_seg(True)(y1, src, dst)
    cnt = cntp.reshape(_NC, _NPAD, 1)
    y2, r2 = _mid(seg1, cnt, r1, W2_l, W2_r, b2r)
    (seg2,) = _make_seg(False)(y2, src, dst)
    out = _fin(seg2, cnt, r2)
    return out
